# baseline (speedup 1.0000x reference)
import jax
import jax.numpy as jnp
from jax import lax
from jax.experimental import pallas as pl
from jax.experimental.pallas import tpu as pltpu
from jax.experimental.pallas import tpu_sc as plsc

NUM_EMBEDDINGS = 1000
EMBED_DIM = 7
CONT_DIM = 6
OUT_DIM = EMBED_DIM + CONT_DIM

NC = 2
NS = 16
NW = NC * NS
LANES = 16

B = 16384
L = 200
TAB_STRIDE = 1024

BW = B // NW
NLB = L // 8
XW = BW // LANES

BWT = 2048


def _sc_body(idx_hbm, tab_hbm, out_hbm, tab_v, idx_v, slab_v,
             sem_idx, sem_out):
  wid = lax.axis_index("s") * NC + lax.axis_index("c")
  b0 = wid * BW

  pltpu.sync_copy(tab_hbm, tab_v)

  def idx_copy(li, s):
    return pltpu.make_async_copy(
        idx_hbm.at[pl.ds(li * 8, 8), pl.ds(b0, BW)], idx_v.at[s],
        sem_idx.at[s])

  def out_copy(li, s):
    return pltpu.make_async_copy(
        slab_v.at[s],
        out_hbm.at[pl.ds(0, EMBED_DIM), pl.ds(li * 8, 8), pl.ds(b0, BW)],
        sem_out.at[s])

  def gather_block(s):
    def gcol(x, _):
      xoff = x * LANES
      for r in range(8):
        iv = idx_v[s, r, pl.ds(xoff, LANES)]
        for c in range(EMBED_DIM):
          vals = plsc.load_gather(tab_v, [iv + (c * TAB_STRIDE)])
          slab_v[s, c, r, pl.ds(xoff, LANES)] = vals
      return 0

    lax.fori_loop(0, XW, gcol, 0)

  idx_copy(0, 0).start()
  for li in (0, 1):
    s = li % 2
    idx_copy(li + 1, 1 - s).start()
    idx_copy(li, s).wait()
    gather_block(s)
    out_copy(li, s).start()

  def pair_body(p, _):
    for s in (0, 1):
      li = p * 2 + s
      out_copy(li, s).wait()
      idx_copy(li + 1, 1 - s).start()
      idx_copy(li, s).wait()
      gather_block(s)
      out_copy(li, s).start()
    return 0

  lax.fori_loop(1, (NLB - 1) // 2, pair_body, 0)

  li = NLB - 1
  out_copy(li, 0).wait()
  idx_copy(li, 0).wait()
  gather_block(0)
  out_copy(li, 0).start()

  out_copy(NLB - 2, 1).wait()
  out_copy(NLB - 1, 0).wait()


def _tc_copy_body(cont_ref, out_prev_ref, out_ref):
  del out_prev_ref
  out_ref[...] = cont_ref[...]


@jax.jit
def _run(idx_t, cont_t, tab_cols):
  mesh = plsc.VectorSubcoreMesh(
      core_axis_name="c", subcore_axis_name="s", num_cores=NC,
      num_subcores=NS)
  sc = pl.kernel(
      _sc_body,
      out_type=jax.ShapeDtypeStruct((OUT_DIM, L, B), jnp.float32),
      mesh=mesh,
      compiler_params=pltpu.CompilerParams(
          needs_layout_passes=False, use_tc_tiling_on_sc=True),
      scratch_types=[
          pltpu.VMEM((EMBED_DIM * TAB_STRIDE,), jnp.float32),
          pltpu.VMEM((2, 8, BW), jnp.int32),
          pltpu.VMEM((2, EMBED_DIM, 8, BW), jnp.float32),
          pltpu.SemaphoreType.DMA((2,)),
          pltpu.SemaphoreType.DMA((2,)),
      ],
  )
  emb_out = sc(idx_t, tab_cols)

  tc = pl.pallas_call(
      _tc_copy_body,
      grid=(CONT_DIM, NLB, B // BWT),
      in_specs=[
          pl.BlockSpec((1, 8, BWT), lambda j, l, b: (j, l, b)),
          pl.BlockSpec(memory_space=pl.ANY),
      ],
      out_specs=pl.BlockSpec((1, 8, BWT),
                             lambda j, l, b: (j + EMBED_DIM, l, b)),
      out_shape=jax.ShapeDtypeStruct((OUT_DIM, L, B), jnp.float32),
      input_output_aliases={1: 0},
      compiler_params=pltpu.CompilerParams(
          dimension_semantics=("arbitrary", "arbitrary", "arbitrary")),
  )
  return tc(cont_t, emb_out)


def kernel(SidebarAssetName, SidebarContinuous, buildable_embedding_weight):
  idx_t = jnp.transpose(SidebarAssetName.astype(jnp.int32), (1, 0))
  cont_t = jnp.transpose(SidebarContinuous, (2, 1, 0))
  tab_cols = jnp.zeros((EMBED_DIM, TAB_STRIDE), jnp.float32)
  tab_cols = tab_cols.at[:, :NUM_EMBEDDINGS].set(
      buildable_embedding_weight.T).reshape(EMBED_DIM * TAB_STRIDE)
  out = _run(idx_t, cont_t, tab_cols)
  return jnp.transpose(out, (2, 1, 0))


# speedup vs baseline: 3.0333x; 3.0333x over previous
import jax
import jax.numpy as jnp
from jax import lax
from jax.experimental import pallas as pl
from jax.experimental.pallas import tpu as pltpu
from jax.experimental.pallas import tpu_sc as plsc

NUM_EMBEDDINGS = 1000
EMBED_DIM = 7
CONT_DIM = 6
OUT_DIM = EMBED_DIM + CONT_DIM

NC = 2
NS = 16
NW = NC * NS
LANES = 16

B = 16384
L = 200
TAB_STRIDE = 1024

BW = B // NW
NLB = L // 8
XW = BW // LANES

LT = 40


def _sc_body(idx_hbm, tab_hbm, out_hbm, tab_v, idx_v, slab_v,
             sem_idx, sem_out):
  wid = lax.axis_index("s") * NC + lax.axis_index("c")
  b0 = wid * BW

  pltpu.sync_copy(tab_hbm, tab_v)

  def idx_copy(li, s):
    return pltpu.make_async_copy(
        idx_hbm.at[pl.ds(li * 8, 8), pl.ds(b0, BW)], idx_v.at[s],
        sem_idx.at[s])

  def out_copy(li, s):
    return pltpu.make_async_copy(
        slab_v.at[s],
        out_hbm.at[pl.ds(0, EMBED_DIM), pl.ds(li * 8, 8), pl.ds(b0, BW)],
        sem_out.at[s])

  def gather_block(s):
    def gcol(x, _):
      xoff = x * LANES
      for r in range(8):
        iv = idx_v[s, r, pl.ds(xoff, LANES)]
        for c in range(EMBED_DIM):
          vals = plsc.load_gather(tab_v, [iv + (c * TAB_STRIDE)])
          slab_v[s, c, r, pl.ds(xoff, LANES)] = vals
      return 0

    lax.fori_loop(0, XW, gcol, 0)

  idx_copy(0, 0).start()
  for li in (0, 1):
    s = li % 2
    idx_copy(li + 1, 1 - s).start()
    idx_copy(li, s).wait()
    gather_block(s)
    out_copy(li, s).start()

  def pair_body(p, _):
    for s in (0, 1):
      li = p * 2 + s
      out_copy(li, s).wait()
      idx_copy(li + 1, 1 - s).start()
      idx_copy(li, s).wait()
      gather_block(s)
      out_copy(li, s).start()
    return 0

  lax.fori_loop(1, (NLB - 1) // 2, pair_body, 0)

  li = NLB - 1
  out_copy(li, 0).wait()
  idx_copy(li, 0).wait()
  gather_block(0)
  out_copy(li, 0).start()

  out_copy(NLB - 2, 1).wait()
  out_copy(NLB - 1, 0).wait()


def _tc_copy_body(cont_ref, out_prev_ref, out_ref):
  del out_prev_ref
  out_ref[...] = cont_ref[...]


@jax.jit
def _run(idx_t, cont_t, tab_cols):
  mesh = plsc.VectorSubcoreMesh(
      core_axis_name="c", subcore_axis_name="s", num_cores=NC,
      num_subcores=NS)
  sc = pl.kernel(
      _sc_body,
      out_type=jax.ShapeDtypeStruct((OUT_DIM, L, B), jnp.float32),
      mesh=mesh,
      compiler_params=pltpu.CompilerParams(
          needs_layout_passes=False, use_tc_tiling_on_sc=True),
      scratch_types=[
          pltpu.VMEM((EMBED_DIM * TAB_STRIDE,), jnp.float32),
          pltpu.VMEM((2, 8, BW), jnp.int32),
          pltpu.VMEM((2, EMBED_DIM, 8, BW), jnp.float32),
          pltpu.SemaphoreType.DMA((2,)),
          pltpu.SemaphoreType.DMA((2,)),
      ],
  )
  emb_out = sc(idx_t, tab_cols)

  tc = pl.pallas_call(
      _tc_copy_body,
      grid=(CONT_DIM, L // LT),
      in_specs=[
          pl.BlockSpec((1, LT, B), lambda j, l: (j, l, 0)),
          pl.BlockSpec(memory_space=pl.ANY),
      ],
      out_specs=pl.BlockSpec((1, LT, B),
                             lambda j, l: (j + EMBED_DIM, l, 0)),
      out_shape=jax.ShapeDtypeStruct((OUT_DIM, L, B), jnp.float32),
      input_output_aliases={1: 0},
      compiler_params=pltpu.CompilerParams(
          dimension_semantics=("arbitrary", "arbitrary")),
  )
  return tc(cont_t, emb_out)


def kernel(SidebarAssetName, SidebarContinuous, buildable_embedding_weight):
  idx_t = jnp.transpose(SidebarAssetName.astype(jnp.int32), (1, 0))
  cont_t = jnp.transpose(SidebarContinuous, (2, 1, 0))
  tab_cols = jnp.zeros((EMBED_DIM, TAB_STRIDE), jnp.float32)
  tab_cols = tab_cols.at[:, :NUM_EMBEDDINGS].set(
      buildable_embedding_weight.T).reshape(EMBED_DIM * TAB_STRIDE)
  out = _run(idx_t, cont_t, tab_cols)
  return jnp.transpose(out, (2, 1, 0))


# speedup vs baseline: 5.6064x; 1.8483x over previous
import jax
import jax.numpy as jnp
from jax import lax
from jax.experimental import pallas as pl
from jax.experimental.pallas import tpu as pltpu
from jax.experimental.pallas import tpu_sc as plsc

NUM_EMBEDDINGS = 1000
EMBED_DIM = 7
CONT_DIM = 6
OUT_DIM = EMBED_DIM + CONT_DIM

NC = 2
NS = 16
NW = NC * NS
LANES = 16

B = 16384
L = 200
TAB_STRIDE = 1024

BW = B // NW
NLB = L // 8
XW = BW // LANES

LT = 40


def _sc_body(idx_hbm, tab_hbm, out_hbm, tab_v, idx_v, slab_v,
             sem_idx, sem_out):
  wid = lax.axis_index("s") * NC + lax.axis_index("c")
  b0 = wid * BW

  pltpu.sync_copy(tab_hbm, tab_v)

  def idx_copy(li, s):
    return pltpu.make_async_copy(
        idx_hbm.at[pl.ds(li * 8, 8), pl.ds(b0, BW)], idx_v.at[s],
        sem_idx.at[s])

  def out_copy(li, s):
    return pltpu.make_async_copy(
        slab_v.at[s],
        out_hbm.at[pl.ds(0, EMBED_DIM), pl.ds(li * 8, 8), pl.ds(b0, BW)],
        sem_out.at[s])

  def gather_block(s):
    @plsc.parallel_loop(0, XW * LANES, step=LANES, unroll=2)
    def gcol(xoff):
      for r in range(8):
        iv = idx_v[s, r, pl.ds(xoff, LANES)]
        for c in range(EMBED_DIM):
          vals = plsc.load_gather(tab_v, [iv + (c * TAB_STRIDE)])
          slab_v[s, c, r, pl.ds(xoff, LANES)] = vals

  idx_copy(0, 0).start()
  for li in (0, 1):
    s = li % 2
    idx_copy(li + 1, 1 - s).start()
    idx_copy(li, s).wait()
    gather_block(s)
    out_copy(li, s).start()

  def pair_body(p, _):
    for s in (0, 1):
      li = p * 2 + s
      out_copy(li, s).wait()
      idx_copy(li + 1, 1 - s).start()
      idx_copy(li, s).wait()
      gather_block(s)
      out_copy(li, s).start()
    return 0

  lax.fori_loop(1, (NLB - 1) // 2, pair_body, 0)

  li = NLB - 1
  out_copy(li, 0).wait()
  idx_copy(li, 0).wait()
  gather_block(0)
  out_copy(li, 0).start()

  out_copy(NLB - 2, 1).wait()
  out_copy(NLB - 1, 0).wait()


def _tc_copy_body(cont_ref, out_prev_ref, out_ref):
  del out_prev_ref
  out_ref[...] = cont_ref[...]


@jax.jit
def _run(idx_t, cont_t, tab_cols):
  mesh = plsc.VectorSubcoreMesh(
      core_axis_name="c", subcore_axis_name="s", num_cores=NC,
      num_subcores=NS)
  sc = pl.kernel(
      _sc_body,
      out_type=jax.ShapeDtypeStruct((OUT_DIM, L, B), jnp.float32),
      mesh=mesh,
      compiler_params=pltpu.CompilerParams(
          needs_layout_passes=False, use_tc_tiling_on_sc=True),
      scratch_types=[
          pltpu.VMEM((EMBED_DIM * TAB_STRIDE,), jnp.float32),
          pltpu.VMEM((2, 8, BW), jnp.int32),
          pltpu.VMEM((2, EMBED_DIM, 8, BW), jnp.float32),
          pltpu.SemaphoreType.DMA((2,)),
          pltpu.SemaphoreType.DMA((2,)),
      ],
  )
  emb_out = sc(idx_t, tab_cols)

  tc = pl.pallas_call(
      _tc_copy_body,
      grid=(CONT_DIM, L // LT),
      in_specs=[
          pl.BlockSpec((1, LT, B), lambda j, l: (j, l, 0)),
          pl.BlockSpec(memory_space=pl.ANY),
      ],
      out_specs=pl.BlockSpec((1, LT, B),
                             lambda j, l: (j + EMBED_DIM, l, 0)),
      out_shape=jax.ShapeDtypeStruct((OUT_DIM, L, B), jnp.float32),
      input_output_aliases={1: 0},
      compiler_params=pltpu.CompilerParams(
          dimension_semantics=("arbitrary", "arbitrary")),
  )
  return tc(cont_t, emb_out)


def kernel(SidebarAssetName, SidebarContinuous, buildable_embedding_weight):
  idx_t = jnp.transpose(SidebarAssetName.astype(jnp.int32), (1, 0))
  cont_t = jnp.transpose(SidebarContinuous, (2, 1, 0))
  tab_cols = jnp.zeros((EMBED_DIM, TAB_STRIDE), jnp.float32)
  tab_cols = tab_cols.at[:, :NUM_EMBEDDINGS].set(
      buildable_embedding_weight.T).reshape(EMBED_DIM * TAB_STRIDE)
  out = _run(idx_t, cont_t, tab_cols)
  return jnp.transpose(out, (2, 1, 0))


# speedup vs baseline: 5.9231x; 1.0565x over previous
import jax
import jax.numpy as jnp
from jax import lax
from jax.experimental import pallas as pl
from jax.experimental.pallas import tpu as pltpu
from jax.experimental.pallas import tpu_sc as plsc

NUM_EMBEDDINGS = 1000
EMBED_DIM = 7
CONT_DIM = 6
OUT_DIM = EMBED_DIM + CONT_DIM

NC = 2
NS = 16
NW = NC * NS
LANES = 16

B = 16384
L = 200
TAB_STRIDE = 1024

BW = B // NW
NLB = L // 8
XW = BW // LANES

LT = 40


def _sc_body(idx_hbm, tab_hbm, out_hbm, tab_v, idx_v, slab_v,
             sem_idx, sem_out):
  wid = lax.axis_index("s") * NC + lax.axis_index("c")
  b0 = wid * BW

  pltpu.sync_copy(tab_hbm, tab_v)

  def idx_copy(li, s):
    return pltpu.make_async_copy(
        idx_hbm.at[pl.ds(li * 8, 8), pl.ds(b0, BW)], idx_v.at[s],
        sem_idx.at[s])

  def out_copy(li, s):
    return pltpu.make_async_copy(
        slab_v.at[s],
        out_hbm.at[pl.ds(0, EMBED_DIM), pl.ds(li * 8, 8), pl.ds(b0, BW)],
        sem_out.at[s])

  def gather_block(s):
    @plsc.parallel_loop(0, XW * LANES, step=LANES, unroll=4)
    def gcol(xoff):
      for r in range(8):
        iv = idx_v[s, r, pl.ds(xoff, LANES)]
        for c in range(EMBED_DIM):
          vals = plsc.load_gather(tab_v, [iv + (c * TAB_STRIDE)])
          slab_v[s, c, r, pl.ds(xoff, LANES)] = vals

  idx_copy(0, 0).start()
  for li in (0, 1):
    s = li % 2
    idx_copy(li + 1, 1 - s).start()
    idx_copy(li, s).wait()
    gather_block(s)
    out_copy(li, s).start()

  def pair_body(p, _):
    for s in (0, 1):
      li = p * 2 + s
      out_copy(li, s).wait()
      idx_copy(li + 1, 1 - s).start()
      idx_copy(li, s).wait()
      gather_block(s)
      out_copy(li, s).start()
    return 0

  lax.fori_loop(1, (NLB - 1) // 2, pair_body, 0)

  li = NLB - 1
  out_copy(li, 0).wait()
  idx_copy(li, 0).wait()
  gather_block(0)
  out_copy(li, 0).start()

  out_copy(NLB - 2, 1).wait()
  out_copy(NLB - 1, 0).wait()


def _tc_copy_body(cont_ref, out_prev_ref, out_ref):
  del out_prev_ref
  out_ref[...] = cont_ref[...]


@jax.jit
def _run(idx_t, cont_t, tab_cols):
  mesh = plsc.VectorSubcoreMesh(
      core_axis_name="c", subcore_axis_name="s", num_cores=NC,
      num_subcores=NS)
  sc = pl.kernel(
      _sc_body,
      out_type=jax.ShapeDtypeStruct((OUT_DIM, L, B), jnp.float32),
      mesh=mesh,
      compiler_params=pltpu.CompilerParams(
          needs_layout_passes=False, use_tc_tiling_on_sc=True),
      scratch_types=[
          pltpu.VMEM((EMBED_DIM * TAB_STRIDE,), jnp.float32),
          pltpu.VMEM((2, 8, BW), jnp.int32),
          pltpu.VMEM((2, EMBED_DIM, 8, BW), jnp.float32),
          pltpu.SemaphoreType.DMA((2,)),
          pltpu.SemaphoreType.DMA((2,)),
      ],
  )
  emb_out = sc(idx_t, tab_cols)

  tc = pl.pallas_call(
      _tc_copy_body,
      grid=(CONT_DIM, L // LT),
      in_specs=[
          pl.BlockSpec((1, LT, B), lambda j, l: (j, l, 0)),
          pl.BlockSpec(memory_space=pl.ANY),
      ],
      out_specs=pl.BlockSpec((1, LT, B),
                             lambda j, l: (j + EMBED_DIM, l, 0)),
      out_shape=jax.ShapeDtypeStruct((OUT_DIM, L, B), jnp.float32),
      input_output_aliases={1: 0},
      compiler_params=pltpu.CompilerParams(
          dimension_semantics=("arbitrary", "arbitrary")),
  )
  return tc(cont_t, emb_out)


def kernel(SidebarAssetName, SidebarContinuous, buildable_embedding_weight):
  idx_t = jnp.transpose(SidebarAssetName.astype(jnp.int32), (1, 0))
  cont_t = jnp.transpose(SidebarContinuous, (2, 1, 0))
  tab_cols = jnp.zeros((EMBED_DIM, TAB_STRIDE), jnp.float32)
  tab_cols = tab_cols.at[:, :NUM_EMBEDDINGS].set(
      buildable_embedding_weight.T).reshape(EMBED_DIM * TAB_STRIDE)
  out = _run(idx_t, cont_t, tab_cols)
  return jnp.transpose(out, (2, 1, 0))


# speedup vs baseline: 6.0704x; 1.0249x over previous
import jax
import jax.numpy as jnp
from jax import lax
from jax.experimental import pallas as pl
from jax.experimental.pallas import tpu as pltpu
from jax.experimental.pallas import tpu_sc as plsc

NUM_EMBEDDINGS = 1000
EMBED_DIM = 7
CONT_DIM = 6
OUT_DIM = EMBED_DIM + CONT_DIM

NC = 2
NS = 16
NW = NC * NS
LANES = 16

B = 16384
L = 200
TAB_STRIDE = 1024

BW = B // NW
NLB = L // 8
XW = BW // LANES

LT = 200


def _sc_body(idx_hbm, tab_hbm, out_hbm, tab_v, idx_v, slab_v,
             sem_idx, sem_out):
  wid = lax.axis_index("s") * NC + lax.axis_index("c")
  b0 = wid * BW

  pltpu.sync_copy(tab_hbm, tab_v)

  def idx_copy(li, s):
    return pltpu.make_async_copy(
        idx_hbm.at[pl.ds(li * 8, 8), pl.ds(b0, BW)], idx_v.at[s],
        sem_idx.at[s])

  def out_copy(li, s):
    return pltpu.make_async_copy(
        slab_v.at[s],
        out_hbm.at[pl.ds(0, EMBED_DIM), pl.ds(li * 8, 8), pl.ds(b0, BW)],
        sem_out.at[s])

  def gather_block(s):
    @plsc.parallel_loop(0, XW * LANES, step=LANES, unroll=4)
    def gcol(xoff):
      for r in range(8):
        iv = idx_v[s, r, pl.ds(xoff, LANES)]
        for c in range(EMBED_DIM):
          vals = plsc.load_gather(tab_v, [iv + (c * TAB_STRIDE)])
          slab_v[s, c, r, pl.ds(xoff, LANES)] = vals

  idx_copy(0, 0).start()
  for li in (0, 1):
    s = li % 2
    idx_copy(li + 1, 1 - s).start()
    idx_copy(li, s).wait()
    gather_block(s)
    out_copy(li, s).start()

  def pair_body(p, _):
    for s in (0, 1):
      li = p * 2 + s
      out_copy(li, s).wait()
      idx_copy(li + 1, 1 - s).start()
      idx_copy(li, s).wait()
      gather_block(s)
      out_copy(li, s).start()
    return 0

  lax.fori_loop(1, (NLB - 1) // 2, pair_body, 0)

  li = NLB - 1
  out_copy(li, 0).wait()
  idx_copy(li, 0).wait()
  gather_block(0)
  out_copy(li, 0).start()

  out_copy(NLB - 2, 1).wait()
  out_copy(NLB - 1, 0).wait()


def _tc_copy_body(cont_ref, out_prev_ref, out_ref):
  del out_prev_ref
  out_ref[...] = cont_ref[...]


@jax.jit
def _run(idx_t, cont_t, tab_cols):
  mesh = plsc.VectorSubcoreMesh(
      core_axis_name="c", subcore_axis_name="s", num_cores=NC,
      num_subcores=NS)
  sc = pl.kernel(
      _sc_body,
      out_type=jax.ShapeDtypeStruct((OUT_DIM, L, B), jnp.float32),
      mesh=mesh,
      compiler_params=pltpu.CompilerParams(
          needs_layout_passes=False, use_tc_tiling_on_sc=True),
      scratch_types=[
          pltpu.VMEM((EMBED_DIM * TAB_STRIDE,), jnp.float32),
          pltpu.VMEM((2, 8, BW), jnp.int32),
          pltpu.VMEM((2, EMBED_DIM, 8, BW), jnp.float32),
          pltpu.SemaphoreType.DMA((2,)),
          pltpu.SemaphoreType.DMA((2,)),
      ],
  )
  emb_out = sc(idx_t, tab_cols)

  tc = pl.pallas_call(
      _tc_copy_body,
      grid=(CONT_DIM, L // LT),
      in_specs=[
          pl.BlockSpec((1, LT, B), lambda j, l: (j, l, 0)),
          pl.BlockSpec(memory_space=pl.ANY),
      ],
      out_specs=pl.BlockSpec((1, LT, B),
                             lambda j, l: (j + EMBED_DIM, l, 0)),
      out_shape=jax.ShapeDtypeStruct((OUT_DIM, L, B), jnp.float32),
      input_output_aliases={1: 0},
      compiler_params=pltpu.CompilerParams(
          dimension_semantics=("arbitrary", "arbitrary")),
  )
  return tc(cont_t, emb_out)


def kernel(SidebarAssetName, SidebarContinuous, buildable_embedding_weight):
  idx_t = jnp.transpose(SidebarAssetName.astype(jnp.int32), (1, 0))
  cont_t = jnp.transpose(SidebarContinuous, (2, 1, 0))
  tab_cols = jnp.zeros((EMBED_DIM, TAB_STRIDE), jnp.float32)
  tab_cols = tab_cols.at[:, :NUM_EMBEDDINGS].set(
      buildable_embedding_weight.T).reshape(EMBED_DIM * TAB_STRIDE)
  out = _run(idx_t, cont_t, tab_cols)
  return jnp.transpose(out, (2, 1, 0))
